# Initial kernel scaffold; baseline (speedup 1.0000x reference)
import jax
import jax.numpy as jnp
from jax.experimental import pallas as pl


def kernel(x, edge_index, W1, att_src1, att_dst1, b1, W2, att_src2, att_dst2, b2):
    raise NotImplementedError("write your pallas kernel here")


# speedup vs baseline: 24.1875x; 24.1875x over previous
import functools

import jax
import jax.numpy as jnp
from jax import lax
from jax.experimental import pallas as pl
from jax.experimental.pallas import tpu as pltpu
from jax.experimental.pallas import tpu_sc as plsc

_N = 10000
_E = 320000
_H = 8
_NSUB = 16
_NCORE = 2
_CH = 128
_CHUNKS = 160
_EPW = _CHUNKS * _CH
_EPAD = _EPW * _NSUB
_ROWS_A = 640
_ROWS_LAST = _N - 15 * _ROWS_A
_NT = _N + 16
_LANES = 16


def _take16(v, idx):
    return jnp.take_along_axis(v, idx, axis=0, mode="promise_in_bounds")


def _sc_edge_layer(srcs2, dsts2, acomb, xlq, cw, cph, npass):
    nvr = cw // _LANES
    assert cw % cph == 0
    hspan = cw // cph
    mesh = plsc.VectorSubcoreMesh(core_axis_name="c", subcore_axis_name="s")

    @functools.partial(
        pl.kernel,
        out_type=(
            jax.ShapeDtypeStruct((_NCORE * npass, _N, cw), jnp.float32),
            jax.ShapeDtypeStruct((_NCORE, _N, 16), jnp.float32),
        ),
        mesh=mesh,
        compiler_params=pltpu.CompilerParams(use_tc_tiling_on_sc=False),
        scratch_types=[
            pltpu.VMEM((_CHUNKS, _CH), jnp.int32),
            pltpu.VMEM((_CHUNKS, _CH), jnp.int32),
            pltpu.VMEM((_CH, 16), jnp.float32),
            pltpu.VMEM((_CH, 16), jnp.float32),
            pltpu.VMEM((_CH, 16), jnp.float32),
            pltpu.VMEM((_CH, cw), jnp.float32),
            pltpu.SemaphoreType.DMA,
            pltpu.SemaphoreType.DMA,
            pltpu.SemaphoreType.DMA,
            pltpu.VMEM_SHARED((_NT, cw), jnp.float32),
            pltpu.VMEM_SHARED((_NT, 16), jnp.float32),
        ],
    )
    def body(srcs_h, dsts_h, acomb_h, xlq_h, accs_h, dens_h,
             src_v, dst_v, a1_v, a2_v, ex_v, xl_v, sem1, sem2, sem3,
             acc_sh, den_sh):
        c = lax.axis_index("c")
        s = lax.axis_index("s")
        row0 = s * _ROWS_A

        def zrow(i, _):
            for k in range(nvr):
                xl_v[i, pl.ds(k * _LANES, _LANES)] = jnp.zeros(
                    (_LANES,), jnp.float32)
            ex_v[i] = jnp.zeros((_LANES,), jnp.float32)
            return _

        def zero_rows(nrows, with_den):
            nfull, rem = divmod(nrows, _CH)
            for r in range(nfull):
                pltpu.sync_copy(xl_v, acc_sh.at[pl.ds(row0 + r * _CH, _CH)])
                if with_den:
                    pltpu.sync_copy(ex_v,
                                    den_sh.at[pl.ds(row0 + r * _CH, _CH)])
            if rem:
                pltpu.sync_copy(xl_v.at[pl.ds(0, rem)],
                                acc_sh.at[pl.ds(row0 + nfull * _CH, rem)])
                if with_den:
                    pltpu.sync_copy(ex_v.at[pl.ds(0, rem)],
                                    den_sh.at[pl.ds(row0 + nfull * _CH, rem)])

        def run_chunks(span, first):
            iot = lax.iota(jnp.int32, _LANES)
            shuf = (iot & 7) + 8
            lane8 = iot < 8

            def fdiv(xv):
                if cph == 16:
                    return xv >> 4
                assert cph == 40
                return (xv * 205) >> 13

            pats = [fdiv(iot + k * _LANES) + span * hspan
                    for k in range(nvr)]

            def chunk(j, _):
                srow = src_v.at[j]
                drow = dst_v.at[j]
                cp1 = pltpu.async_copy(acomb_h.at[srow], a1_v, sem1)
                cp2 = pltpu.async_copy(acomb_h.at[drow], a2_v, sem2)
                cp3 = pltpu.async_copy(xlq_h.at[span].at[srow], xl_v, sem3)
                cp1.wait()
                cp2.wait()
                cp3.wait()

                def edge(e, _):
                    a = a1_v[e] + _take16(a2_v[e], shuf)
                    a = jnp.where(a >= 0, a, 0.2 * a)
                    exr = jnp.where(lane8, jnp.exp(a), 0.0)
                    ex_v[e] = exr
                    for k in range(nvr):
                        sl = pl.ds(k * _LANES, _LANES)
                        xl_v[e, sl] = xl_v[e, sl] * _take16(exr, pats[k])
                    return _

                lax.fori_loop(0, _CH, edge, None)
                if first:
                    pltpu.sync_copy(ex_v, den_sh.at[drow], add=True)
                pltpu.sync_copy(xl_v, acc_sh.at[drow], add=True)
                return _

            lax.fori_loop(0, _CHUNKS, chunk, None)

        pltpu.sync_copy(srcs_h.at[pl.ds(s * _CHUNKS, _CHUNKS)], src_v)
        pltpu.sync_copy(dsts_h.at[pl.ds(s * _CHUNKS, _CHUNKS)], dst_v)

        for p in range(npass):
            first = p == 0
            lax.fori_loop(0, _CH, zrow, None)
            pl.when(s < _NSUB - 1)(
                functools.partial(zero_rows, _ROWS_A, first))
            pl.when(s == _NSUB - 1)(
                functools.partial(zero_rows, _ROWS_LAST, first))
            plsc.subcore_barrier()
            pl.when(c == 0)(functools.partial(run_chunks, p, first))
            pl.when(c == 1)(functools.partial(run_chunks, npass + p, first))
            plsc.subcore_barrier()

            def publish(nrows, first=first):
                pltpu.sync_copy(
                    acc_sh.at[pl.ds(row0, nrows)],
                    accs_h.at[c * npass + p, pl.ds(row0, nrows)])
                if first:
                    pltpu.sync_copy(den_sh.at[pl.ds(row0, nrows)],
                                    dens_h.at[c, pl.ds(row0, nrows)])

            pl.when(s < _NSUB - 1)(functools.partial(publish, _ROWS_A))
            pl.when(s == _NSUB - 1)(functools.partial(publish, _ROWS_LAST))

    return body(srcs2, dsts2, acomb, xlq)


_BLK = 1000
_GRID = _N // _BLK


def _tc_prep(x, W, AC, hc, nspan):
    cw = hc // nspan

    def body(x_ref, w_ref, ac_ref, xlh_ref, ac_out_ref):
        xl = jnp.dot(x_ref[...], w_ref[...],
                     preferred_element_type=jnp.float32)
        for q in range(nspan):
            xlh_ref[q] = xl[:, q * cw:(q + 1) * cw]
        ac_out_ref[...] = jnp.dot(xl, ac_ref[...],
                                  preferred_element_type=jnp.float32)

    fin = x.shape[1]
    return pl.pallas_call(
        body,
        grid=(_GRID,),
        in_specs=[
            pl.BlockSpec((_BLK, fin), lambda i: (i, 0)),
            pl.BlockSpec((fin, hc), lambda i: (0, 0)),
            pl.BlockSpec((hc, 16), lambda i: (0, 0)),
        ],
        out_specs=[
            pl.BlockSpec((nspan, _BLK, cw), lambda i: (0, i, 0)),
            pl.BlockSpec((_BLK, 16), lambda i: (i, 0)),
        ],
        out_shape=[
            jax.ShapeDtypeStruct((nspan, _N, cw), jnp.float32),
            jax.ShapeDtypeStruct((_N, 16), jnp.float32),
        ],
    )(x, W, AC)


def _tc_mid(accs, dens, Eh, b, W2, AC2, hc1, hc2, nspan1, nspan2):
    cw2 = hc2 // nspan2

    def body(acc_ref, den_ref, eh_ref, b_ref, w_ref, ac_ref,
             xlh_ref, ac_out_ref):
        cat = jnp.concatenate([acc_ref[q] for q in range(nspan1)], axis=1)
        dexp = jnp.dot(den_ref[0], eh_ref[...],
                       preferred_element_type=jnp.float32)
        h1 = jnp.maximum(cat / (dexp + 1e-16) + b_ref[0:1, :], 0.0)
        xl2 = jnp.dot(h1, w_ref[...], preferred_element_type=jnp.float32)
        for q in range(nspan2):
            xlh_ref[q] = xl2[:, q * cw2:(q + 1) * cw2]
        ac_out_ref[...] = jnp.dot(xl2, ac_ref[...],
                                  preferred_element_type=jnp.float32)

    return pl.pallas_call(
        body,
        grid=(_GRID,),
        in_specs=[
            pl.BlockSpec((nspan1, _BLK, hc1 // nspan1), lambda i: (0, i, 0)),
            pl.BlockSpec((1, _BLK, 16), lambda i: (0, i, 0)),
            pl.BlockSpec((16, hc1), lambda i: (0, 0)),
            pl.BlockSpec((8, hc1), lambda i: (0, 0)),
            pl.BlockSpec((hc1, hc2), lambda i: (0, 0)),
            pl.BlockSpec((hc2, 16), lambda i: (0, 0)),
        ],
        out_specs=[
            pl.BlockSpec((nspan2, _BLK, cw2), lambda i: (0, i, 0)),
            pl.BlockSpec((_BLK, 16), lambda i: (i, 0)),
        ],
        out_shape=[
            jax.ShapeDtypeStruct((nspan2, _N, cw2), jnp.float32),
            jax.ShapeDtypeStruct((_N, 16), jnp.float32),
        ],
    )(accs, dens, Eh, b, W2, AC2)


def _tc_fin(accs, dens, Eh, b, hc, nspan):
    def body(acc_ref, den_ref, eh_ref, b_ref, out_ref):
        cat = jnp.concatenate([acc_ref[q] for q in range(nspan)], axis=1)
        dexp = jnp.dot(den_ref[0], eh_ref[...],
                       preferred_element_type=jnp.float32)
        out_ref[...] = cat / (dexp + 1e-16) + b_ref[0:1, :]

    return pl.pallas_call(
        body,
        grid=(_GRID,),
        in_specs=[
            pl.BlockSpec((nspan, _BLK, hc // nspan), lambda i: (0, i, 0)),
            pl.BlockSpec((1, _BLK, 16), lambda i: (0, i, 0)),
            pl.BlockSpec((16, hc), lambda i: (0, 0)),
            pl.BlockSpec((8, hc), lambda i: (0, 0)),
        ],
        out_specs=pl.BlockSpec((_BLK, hc), lambda i: (i, 0)),
        out_shape=jax.ShapeDtypeStruct((_N, hc), jnp.float32),
    )(accs, dens, Eh, b)


def _att_matrix(att_src, att_dst):
    h, cdim = att_src.shape[1], att_src.shape[2]
    eye = jnp.eye(h, dtype=jnp.float32)
    msrc = (att_src[0][:, :, None] * eye[:, None, :]).reshape(h * cdim, h)
    mdst = (att_dst[0][:, :, None] * eye[:, None, :]).reshape(h * cdim, h)
    return jnp.concatenate([msrc, mdst], axis=1)


def _head_expander(hc):
    top = jnp.kron(jnp.eye(_H, dtype=jnp.float32),
                   jnp.ones((1, hc // _H), jnp.float32))
    return jnp.concatenate([top, jnp.zeros((_H, hc), jnp.float32)], axis=0)


def kernel(x, edge_index, W1, att_src1, att_dst1, b1,
           W2, att_src2, att_dst2, b2):
    hc1 = W1.shape[1]
    hc2 = W2.shape[1]
    c1 = hc1 // _H
    c2 = hc2 // _H

    pad = _EPAD - _E
    fill = jnp.full((pad,), _N, jnp.int32)
    srcs2 = jnp.concatenate([edge_index[0], fill]).reshape(-1, _CH)
    dsts2 = jnp.concatenate([edge_index[1], fill]).reshape(-1, _CH)

    AC1 = _att_matrix(att_src1, att_dst1)
    AC2 = _att_matrix(att_src2, att_dst2)
    Eh1 = _head_expander(hc1)
    Eh2 = _head_expander(hc2)
    b1r = jnp.broadcast_to(b1.reshape(1, hc1), (8, hc1))
    b2r = jnp.broadcast_to(b2.reshape(1, hc2), (8, hc2))

    def extend(acomb, xlq):
        nspan, _, cw = xlq.shape
        acomb_e = jnp.concatenate(
            [acomb, jnp.full((_NT - _N, 16), -1e30, jnp.float32)], axis=0)
        xlq_e = jnp.concatenate(
            [xlq, jnp.zeros((nspan, _NT - _N, cw), jnp.float32)], axis=1)
        return acomb_e, xlq_e

    np1, np2 = 1, 2
    xlq1, acomb1 = _tc_prep(x, W1, AC1, hc1, 2 * np1)
    acomb1, xlq1 = extend(acomb1, xlq1)
    accs1, dens1 = _sc_edge_layer(srcs2, dsts2, acomb1, xlq1,
                                  hc1 // (2 * np1), c1, np1)
    xlq2, acomb2 = _tc_mid(accs1, dens1, Eh1, b1r, W2, AC2, hc1, hc2,
                           2 * np1, 2 * np2)
    acomb2, xlq2 = extend(acomb2, xlq2)
    accs2, dens2 = _sc_edge_layer(srcs2, dsts2, acomb2, xlq2,
                                  hc2 // (2 * np2), c2, np2)
    return _tc_fin(accs2, dens2, Eh2, b2r, hc2, 2 * np2)


# speedup vs baseline: 26.1224x; 1.0800x over previous
import functools

import jax
import jax.numpy as jnp
from jax import lax
from jax.experimental import pallas as pl
from jax.experimental.pallas import tpu as pltpu
from jax.experimental.pallas import tpu_sc as plsc

_N = 10000
_E = 320000
_H = 8
_NSUB = 16
_NCORE = 2
_CH = 128
_CHUNKS = 160
_EPW = _CHUNKS * _CH
_EPAD = _EPW * _NSUB
_ROWS_A = 640
_ROWS_LAST = _N - 15 * _ROWS_A
_NT = _N + 16
_NB = 2
_LANES = 16


def _take16(v, idx):
    return jnp.take_along_axis(v, idx, axis=0, mode="promise_in_bounds")


def _sc_edge_layer(srcs2, dsts2, acomb, xlq, cw, cph, npass):
    if cw % _LANES == 0:
        offs = list(range(0, cw, _LANES))
    else:
        offs = list(range(0, cw - _LANES, _LANES)) + [cw - _LANES]
    mesh = plsc.VectorSubcoreMesh(core_axis_name="c", subcore_axis_name="s")

    @functools.partial(
        pl.kernel,
        out_type=(
            jax.ShapeDtypeStruct((_NCORE * npass, _N, cw), jnp.float32),
            jax.ShapeDtypeStruct((1, _N, 16), jnp.float32),
        ),
        mesh=mesh,
        compiler_params=pltpu.CompilerParams(use_tc_tiling_on_sc=False),
        scratch_types=[
            pltpu.VMEM((_CHUNKS, _CH), jnp.int32),
            pltpu.VMEM((_CHUNKS, _CH), jnp.int32),
            pltpu.VMEM((_NB, _CH, 16), jnp.float32),
            pltpu.VMEM((_NB, _CH, 16), jnp.float32),
            pltpu.VMEM((_NB, _CH, 16), jnp.float32),
            pltpu.VMEM((_NB, _CH, cw), jnp.float32),
        ] + [pltpu.SemaphoreType.DMA] * _NB + [
            pltpu.VMEM_SHARED((_NT, cw), jnp.float32),
            pltpu.VMEM_SHARED((_NT, 16), jnp.float32),
        ],
    )
    def body(srcs_h, dsts_h, acomb_h, xlq_h, accs_h, dens_h,
             src_v, dst_v, a1_v, a2_v, ex_v, xl_v,
             g0, g1,
             acc_sh, den_sh):
        gsem = [g0, g1]
        c = lax.axis_index("c")
        s = lax.axis_index("s")
        row0 = s * _ROWS_A

        def zrow(i, _):
            for o in offs:
                xl_v[0, i, pl.ds(o, _LANES)] = jnp.zeros(
                    (_LANES,), jnp.float32)
            ex_v[0, i] = jnp.zeros((_LANES,), jnp.float32)
            return _

        def zero_rows(nrows, with_den):
            zx = xl_v.at[0]
            ze = ex_v.at[0]
            nfull, rem = divmod(nrows, _CH)
            for r in range(nfull):
                pltpu.sync_copy(zx, acc_sh.at[pl.ds(row0 + r * _CH, _CH)])
                if with_den:
                    pltpu.sync_copy(ze,
                                    den_sh.at[pl.ds(row0 + r * _CH, _CH)])
            if rem:
                pltpu.sync_copy(zx.at[pl.ds(0, rem)],
                                acc_sh.at[pl.ds(row0 + nfull * _CH, rem)])
                if with_den:
                    pltpu.sync_copy(ze.at[pl.ds(0, rem)],
                                    den_sh.at[pl.ds(row0 + nfull * _CH, rem)])

        def run_chunks(span, first):
            iot = lax.iota(jnp.int32, _LANES)
            shuf = (iot & 7) + 8
            lane8 = iot < 8

            def fdiv(xv):
                if cph == 16:
                    return xv >> 4
                assert cph == 40
                return (xv * 205) >> 13

            pats = {o: fdiv(span * cw + o + iot) for o in offs}

            def issue_gathers(j, b):
                srow = src_v.at[j]
                pltpu.async_copy(acomb_h.at[srow], a1_v.at[b], gsem[b])
                pltpu.async_copy(acomb_h.at[dst_v.at[j]], a2_v.at[b],
                                 gsem[b])
                pltpu.async_copy(xlq_h.at[span].at[srow], xl_v.at[b],
                                 gsem[b])

            def wait_gathers(j, b):
                srow = src_v.at[j]
                pltpu.make_async_copy(
                    acomb_h.at[srow], a1_v.at[b], gsem[b]).wait()
                pltpu.make_async_copy(
                    acomb_h.at[dst_v.at[j]], a2_v.at[b], gsem[b]).wait()
                pltpu.make_async_copy(
                    xlq_h.at[span].at[srow], xl_v.at[b], gsem[b]).wait()

            def scatters(j, b):
                drow = dst_v.at[j]
                if first:
                    pltpu.sync_copy(ex_v.at[b], den_sh.at[drow], add=True)
                pltpu.sync_copy(xl_v.at[b], acc_sh.at[drow], add=True)

            def compute(j, b):
                def edge2(i, _):
                    for u in range(2):
                        e = 2 * i + u
                        a = a1_v[b, e] + _take16(a2_v[b, e], shuf)
                        a = jnp.where(a >= 0, a, 0.2 * a)
                        exr = jnp.where(lane8, jnp.exp(a), 0.0)
                        if first:
                            ex_v[b, e] = exr
                        vals = [xl_v[b, e, pl.ds(o, _LANES)] for o in offs]
                        for o, v in zip(offs, vals):
                            xl_v[b, e, pl.ds(o, _LANES)] = (
                                v * _take16(exr, pats[o]))
                    return _

                lax.fori_loop(0, _CH // 2, edge2, None)

            def phase(j, b):
                wait_gathers(j, b)
                pl.when(j < _CHUNKS - 1)(
                    lambda: issue_gathers(j + 1, 1 - b))
                compute(j, b)
                scatters(j, b)

            issue_gathers(0, 0)

            def pair(g, _):
                phase(2 * g, 0)
                phase(2 * g + 1, 1)
                return _

            lax.fori_loop(0, _CHUNKS // 2, pair, None)

        pltpu.sync_copy(srcs_h.at[pl.ds(s * _CHUNKS, _CHUNKS)], src_v)
        pltpu.sync_copy(dsts_h.at[pl.ds(s * _CHUNKS, _CHUNKS)], dst_v)

        for p in range(npass):
            first = p == 0
            lax.fori_loop(0, _CH, zrow, None)
            pl.when(s < _NSUB - 1)(
                functools.partial(zero_rows, _ROWS_A, first))
            pl.when(s == _NSUB - 1)(
                functools.partial(zero_rows, _ROWS_LAST, first))
            plsc.subcore_barrier()
            pl.when(c == 0)(functools.partial(run_chunks, p, first))
            pl.when(c == 1)(functools.partial(run_chunks, npass + p, False))
            plsc.subcore_barrier()

            def publish(nrows, first=first):
                pltpu.sync_copy(
                    acc_sh.at[pl.ds(row0, nrows)],
                    accs_h.at[c * npass + p, pl.ds(row0, nrows)])

            def publish_den(nrows):
                pltpu.sync_copy(den_sh.at[pl.ds(row0, nrows)],
                                dens_h.at[0, pl.ds(row0, nrows)])

            pl.when(s < _NSUB - 1)(functools.partial(publish, _ROWS_A))
            pl.when(s == _NSUB - 1)(functools.partial(publish, _ROWS_LAST))
            if p == 0:
                pl.when(jnp.logical_and(c == 0, s < _NSUB - 1))(
                    functools.partial(publish_den, _ROWS_A))
                pl.when(jnp.logical_and(c == 0, s == _NSUB - 1))(
                    functools.partial(publish_den, _ROWS_LAST))

    return body(srcs2, dsts2, acomb, xlq)


_BLK = 1000
_GRID = _N // _BLK


def _tc_prep(x, W, AC, hc, nspan):
    cw = hc // nspan

    def body(x_ref, w_ref, ac_ref, xlh_ref, ac_out_ref):
        xl = jnp.dot(x_ref[...], w_ref[...],
                     preferred_element_type=jnp.float32)
        for q in range(nspan):
            xlh_ref[q] = xl[:, q * cw:(q + 1) * cw]
        ac_out_ref[...] = jnp.dot(xl, ac_ref[...],
                                  preferred_element_type=jnp.float32)

    fin = x.shape[1]
    return pl.pallas_call(
        body,
        grid=(_GRID,),
        in_specs=[
            pl.BlockSpec((_BLK, fin), lambda i: (i, 0)),
            pl.BlockSpec((fin, hc), lambda i: (0, 0)),
            pl.BlockSpec((hc, 16), lambda i: (0, 0)),
        ],
        out_specs=[
            pl.BlockSpec((nspan, _BLK, cw), lambda i: (0, i, 0)),
            pl.BlockSpec((_BLK, 16), lambda i: (i, 0)),
        ],
        out_shape=[
            jax.ShapeDtypeStruct((nspan, _N, cw), jnp.float32),
            jax.ShapeDtypeStruct((_N, 16), jnp.float32),
        ],
    )(x, W, AC)


def _tc_mid(accs, dens, Eh, b, W2, AC2, hc1, hc2, nspan1, nspan2):
    cw2 = hc2 // nspan2

    def body(acc_ref, den_ref, eh_ref, b_ref, w_ref, ac_ref,
             xlh_ref, ac_out_ref):
        cat = jnp.concatenate([acc_ref[q] for q in range(nspan1)], axis=1)
        dexp = jnp.dot(den_ref[0], eh_ref[...],
                       preferred_element_type=jnp.float32)
        h1 = jnp.maximum(cat / (dexp + 1e-16) + b_ref[0:1, :], 0.0)
        xl2 = jnp.dot(h1, w_ref[...], preferred_element_type=jnp.float32)
        for q in range(nspan2):
            xlh_ref[q] = xl2[:, q * cw2:(q + 1) * cw2]
        ac_out_ref[...] = jnp.dot(xl2, ac_ref[...],
                                  preferred_element_type=jnp.float32)

    return pl.pallas_call(
        body,
        grid=(_GRID,),
        in_specs=[
            pl.BlockSpec((nspan1, _BLK, hc1 // nspan1), lambda i: (0, i, 0)),
            pl.BlockSpec((1, _BLK, 16), lambda i: (0, i, 0)),
            pl.BlockSpec((16, hc1), lambda i: (0, 0)),
            pl.BlockSpec((8, hc1), lambda i: (0, 0)),
            pl.BlockSpec((hc1, hc2), lambda i: (0, 0)),
            pl.BlockSpec((hc2, 16), lambda i: (0, 0)),
        ],
        out_specs=[
            pl.BlockSpec((nspan2, _BLK, cw2), lambda i: (0, i, 0)),
            pl.BlockSpec((_BLK, 16), lambda i: (i, 0)),
        ],
        out_shape=[
            jax.ShapeDtypeStruct((nspan2, _N, cw2), jnp.float32),
            jax.ShapeDtypeStruct((_N, 16), jnp.float32),
        ],
    )(accs, dens, Eh, b, W2, AC2)


def _tc_fin(accs, dens, Eh, b, hc, nspan):
    def body(acc_ref, den_ref, eh_ref, b_ref, out_ref):
        cat = jnp.concatenate([acc_ref[q] for q in range(nspan)], axis=1)
        dexp = jnp.dot(den_ref[0], eh_ref[...],
                       preferred_element_type=jnp.float32)
        out_ref[...] = cat / (dexp + 1e-16) + b_ref[0:1, :]

    return pl.pallas_call(
        body,
        grid=(_GRID,),
        in_specs=[
            pl.BlockSpec((nspan, _BLK, hc // nspan), lambda i: (0, i, 0)),
            pl.BlockSpec((1, _BLK, 16), lambda i: (0, i, 0)),
            pl.BlockSpec((16, hc), lambda i: (0, 0)),
            pl.BlockSpec((8, hc), lambda i: (0, 0)),
        ],
        out_specs=pl.BlockSpec((_BLK, hc), lambda i: (i, 0)),
        out_shape=jax.ShapeDtypeStruct((_N, hc), jnp.float32),
    )(accs, dens, Eh, b)


def _att_matrix(att_src, att_dst):
    h, cdim = att_src.shape[1], att_src.shape[2]
    eye = jnp.eye(h, dtype=jnp.float32)
    msrc = (att_src[0][:, :, None] * eye[:, None, :]).reshape(h * cdim, h)
    mdst = (att_dst[0][:, :, None] * eye[:, None, :]).reshape(h * cdim, h)
    return jnp.concatenate([msrc, mdst], axis=1)


def _head_expander(hc):
    top = jnp.kron(jnp.eye(_H, dtype=jnp.float32),
                   jnp.ones((1, hc // _H), jnp.float32))
    return jnp.concatenate([top, jnp.zeros((_H, hc), jnp.float32)], axis=0)


def kernel(x, edge_index, W1, att_src1, att_dst1, b1,
           W2, att_src2, att_dst2, b2):
    hc1 = W1.shape[1]
    hc2 = W2.shape[1]
    c1 = hc1 // _H
    c2 = hc2 // _H

    pad = _EPAD - _E
    fill = jnp.full((pad,), _N, jnp.int32)
    srcs2 = jnp.concatenate([edge_index[0], fill]).reshape(-1, _CH)
    dsts2 = jnp.concatenate([edge_index[1], fill]).reshape(-1, _CH)

    AC1 = _att_matrix(att_src1, att_dst1)
    AC2 = _att_matrix(att_src2, att_dst2)
    Eh1 = _head_expander(hc1)
    Eh2 = _head_expander(hc2)
    b1r = jnp.broadcast_to(b1.reshape(1, hc1), (8, hc1))
    b2r = jnp.broadcast_to(b2.reshape(1, hc2), (8, hc2))

    def extend(acomb, xlq):
        nspan, _, cw = xlq.shape
        acomb_e = jnp.concatenate(
            [acomb, jnp.full((_NT - _N, 16), -1e30, jnp.float32)], axis=0)
        xlq_e = jnp.concatenate(
            [xlq, jnp.zeros((nspan, _NT - _N, cw), jnp.float32)], axis=1)
        return acomb_e, xlq_e

    np1, np2 = 2, 4
    xlq1, acomb1 = _tc_prep(x, W1, AC1, hc1, 2 * np1)
    acomb1, xlq1 = extend(acomb1, xlq1)
    accs1, dens1 = _sc_edge_layer(srcs2, dsts2, acomb1, xlq1,
                                  hc1 // (2 * np1), c1, np1)
    xlq2, acomb2 = _tc_mid(accs1, dens1, Eh1, b1r, W2, AC2, hc1, hc2,
                           2 * np1, 2 * np2)
    acomb2, xlq2 = extend(acomb2, xlq2)
    accs2, dens2 = _sc_edge_layer(srcs2, dsts2, acomb2, xlq2,
                                  hc2 // (2 * np2), c2, np2)
    return _tc_fin(accs2, dens2, Eh2, b2r, hc2, 2 * np2)


# speedup vs baseline: 37.2057x; 1.4243x over previous
import functools

import jax
import jax.numpy as jnp
from jax import lax
from jax.experimental import pallas as pl
from jax.experimental.pallas import tpu as pltpu
from jax.experimental.pallas import tpu_sc as plsc

_N = 10000
_E = 320000
_H = 8
_NSUB = 16
_NCORE = 2
_CH = 128
_CHUNKS = 160
_EPW = _CHUNKS * _CH
_EPAD = _EPW * _NSUB
_ROWS_A = 640
_ROWS_LAST = _N - 15 * _ROWS_A
_NT = _N + 16
_NB = 2
_LANES = 16


def _take16(v, idx):
    return jnp.take_along_axis(v, idx, axis=0, mode="promise_in_bounds")


def _sc_edge_layer(srcs2, dsts2, acomb, xlq, cw, cph, npass):
    if cw % _LANES == 0:
        offs = list(range(0, cw, _LANES))
    else:
        offs = list(range(0, cw - _LANES, _LANES)) + [cw - _LANES]
    mesh = plsc.VectorSubcoreMesh(core_axis_name="c", subcore_axis_name="s")

    @functools.partial(
        pl.kernel,
        out_type=(
            jax.ShapeDtypeStruct((_NCORE * npass, _N, cw), jnp.float32),
            jax.ShapeDtypeStruct((1, _N, 16), jnp.float32),
        ),
        mesh=mesh,
        compiler_params=pltpu.CompilerParams(use_tc_tiling_on_sc=False),
        scratch_types=[
            pltpu.VMEM((_CHUNKS, _CH), jnp.int32),
            pltpu.VMEM((_CHUNKS, _CH), jnp.int32),
            pltpu.VMEM((_NB, _CH, 16), jnp.float32),
            pltpu.VMEM((_NB, _CH, 16), jnp.float32),
            pltpu.VMEM((_NB, _CH, 16), jnp.float32),
            pltpu.VMEM((_NB, _CH, cw), jnp.float32),
        ] + [pltpu.SemaphoreType.DMA] * _NB + [
            pltpu.VMEM_SHARED((_NT, cw), jnp.float32),
            pltpu.VMEM_SHARED((_NT, 16), jnp.float32),
        ],
    )
    def body(srcs_h, dsts_h, acomb_h, xlq_h, accs_h, dens_h,
             src_v, dst_v, a1_v, a2_v, ex_v, xl_v,
             g0, g1,
             acc_sh, den_sh):
        gsem = [g0, g1]
        c = lax.axis_index("c")
        s = lax.axis_index("s")
        row0 = s * _ROWS_A

        def zrow(i, _):
            for o in offs:
                xl_v[0, i, pl.ds(o, _LANES)] = jnp.zeros(
                    (_LANES,), jnp.float32)
            ex_v[0, i] = jnp.zeros((_LANES,), jnp.float32)
            return _

        def zero_rows(nrows, with_den):
            zx = xl_v.at[0]
            ze = ex_v.at[0]
            nfull, rem = divmod(nrows, _CH)
            for r in range(nfull):
                pltpu.sync_copy(zx, acc_sh.at[pl.ds(row0 + r * _CH, _CH)])
                if with_den:
                    pltpu.sync_copy(ze,
                                    den_sh.at[pl.ds(row0 + r * _CH, _CH)])
            if rem:
                pltpu.sync_copy(zx.at[pl.ds(0, rem)],
                                acc_sh.at[pl.ds(row0 + nfull * _CH, rem)])
                if with_den:
                    pltpu.sync_copy(ze.at[pl.ds(0, rem)],
                                    den_sh.at[pl.ds(row0 + nfull * _CH, rem)])

        def run_chunks(span, first):
            iot = lax.iota(jnp.int32, _LANES)
            shuf = (iot & 7) + 8
            lane8 = iot < 8

            def fdiv(xv):
                if cph == 16:
                    return xv >> 4
                assert cph == 40
                return (xv * 205) >> 13

            pats = {o: fdiv(span * cw + o + iot) for o in offs}

            def issue_gathers(j, b):
                srow = src_v.at[j]
                pltpu.async_copy(acomb_h.at[srow], a1_v.at[b], gsem[b])
                pltpu.async_copy(acomb_h.at[dst_v.at[j]], a2_v.at[b],
                                 gsem[b])
                pltpu.async_copy(xlq_h.at[span].at[srow], xl_v.at[b],
                                 gsem[b])

            def wait_gathers(j, b):
                srow = src_v.at[j]
                pltpu.make_async_copy(
                    acomb_h.at[srow], a1_v.at[b], gsem[b]).wait()
                pltpu.make_async_copy(
                    acomb_h.at[dst_v.at[j]], a2_v.at[b], gsem[b]).wait()
                pltpu.make_async_copy(
                    xlq_h.at[span].at[srow], xl_v.at[b], gsem[b]).wait()

            def scatters(j, b):
                drow = dst_v.at[j]
                if first:
                    pltpu.sync_copy(ex_v.at[b], den_sh.at[drow], add=True)
                pltpu.sync_copy(xl_v.at[b], acc_sh.at[drow], add=True)

            def compute(j, b):
                @plsc.parallel_loop(0, _CH, 1, unroll=4)
                def _(e):
                    a = a1_v[b, e] + _take16(a2_v[b, e], shuf)
                    a = jnp.where(a >= 0, a, 0.2 * a)
                    exr = jnp.where(lane8, jnp.exp(a), 0.0)
                    if first:
                        ex_v[b, e] = exr
                    vals = [xl_v[b, e, pl.ds(o, _LANES)] for o in offs]
                    for o, v in zip(offs, vals):
                        xl_v[b, e, pl.ds(o, _LANES)] = (
                            v * _take16(exr, pats[o]))

            def phase(j, b):
                wait_gathers(j, b)
                pl.when(j < _CHUNKS - 1)(
                    lambda: issue_gathers(j + 1, 1 - b))
                compute(j, b)
                scatters(j, b)

            issue_gathers(0, 0)

            def pair(g, _):
                phase(2 * g, 0)
                phase(2 * g + 1, 1)
                return _

            lax.fori_loop(0, _CHUNKS // 2, pair, None)

        pltpu.sync_copy(srcs_h.at[pl.ds(s * _CHUNKS, _CHUNKS)], src_v)
        pltpu.sync_copy(dsts_h.at[pl.ds(s * _CHUNKS, _CHUNKS)], dst_v)

        for p in range(npass):
            first = p == 0
            lax.fori_loop(0, _CH, zrow, None)
            pl.when(s < _NSUB - 1)(
                functools.partial(zero_rows, _ROWS_A, first))
            pl.when(s == _NSUB - 1)(
                functools.partial(zero_rows, _ROWS_LAST, first))
            plsc.subcore_barrier()
            pl.when(c == 0)(functools.partial(run_chunks, p, first))
            pl.when(c == 1)(functools.partial(run_chunks, npass + p, False))
            plsc.subcore_barrier()

            def publish(nrows, first=first):
                pltpu.sync_copy(
                    acc_sh.at[pl.ds(row0, nrows)],
                    accs_h.at[c * npass + p, pl.ds(row0, nrows)])

            def publish_den(nrows):
                pltpu.sync_copy(den_sh.at[pl.ds(row0, nrows)],
                                dens_h.at[0, pl.ds(row0, nrows)])

            pl.when(s < _NSUB - 1)(functools.partial(publish, _ROWS_A))
            pl.when(s == _NSUB - 1)(functools.partial(publish, _ROWS_LAST))
            if p == 0:
                pl.when(jnp.logical_and(c == 0, s < _NSUB - 1))(
                    functools.partial(publish_den, _ROWS_A))
                pl.when(jnp.logical_and(c == 0, s == _NSUB - 1))(
                    functools.partial(publish_den, _ROWS_LAST))

    return body(srcs2, dsts2, acomb, xlq)


_BLK = 1000
_GRID = _N // _BLK


def _tc_prep(x, W, AC, hc, nspan):
    cw = hc // nspan

    def body(x_ref, w_ref, ac_ref, xlh_ref, ac_out_ref):
        xl = jnp.dot(x_ref[...], w_ref[...],
                     preferred_element_type=jnp.float32)
        for q in range(nspan):
            xlh_ref[q] = xl[:, q * cw:(q + 1) * cw]
        ac_out_ref[...] = jnp.dot(xl, ac_ref[...],
                                  preferred_element_type=jnp.float32)

    fin = x.shape[1]
    return pl.pallas_call(
        body,
        grid=(_GRID,),
        in_specs=[
            pl.BlockSpec((_BLK, fin), lambda i: (i, 0)),
            pl.BlockSpec((fin, hc), lambda i: (0, 0)),
            pl.BlockSpec((hc, 16), lambda i: (0, 0)),
        ],
        out_specs=[
            pl.BlockSpec((nspan, _BLK, cw), lambda i: (0, i, 0)),
            pl.BlockSpec((_BLK, 16), lambda i: (i, 0)),
        ],
        out_shape=[
            jax.ShapeDtypeStruct((nspan, _N, cw), jnp.float32),
            jax.ShapeDtypeStruct((_N, 16), jnp.float32),
        ],
    )(x, W, AC)


def _tc_mid(accs, dens, Eh, b, W2, AC2, hc1, hc2, nspan1, nspan2):
    cw2 = hc2 // nspan2

    def body(acc_ref, den_ref, eh_ref, b_ref, w_ref, ac_ref,
             xlh_ref, ac_out_ref):
        cat = jnp.concatenate([acc_ref[q] for q in range(nspan1)], axis=1)
        dexp = jnp.dot(den_ref[0], eh_ref[...],
                       preferred_element_type=jnp.float32)
        h1 = jnp.maximum(cat / (dexp + 1e-16) + b_ref[0:1, :], 0.0)
        xl2 = jnp.dot(h1, w_ref[...], preferred_element_type=jnp.float32)
        for q in range(nspan2):
            xlh_ref[q] = xl2[:, q * cw2:(q + 1) * cw2]
        ac_out_ref[...] = jnp.dot(xl2, ac_ref[...],
                                  preferred_element_type=jnp.float32)

    return pl.pallas_call(
        body,
        grid=(_GRID,),
        in_specs=[
            pl.BlockSpec((nspan1, _BLK, hc1 // nspan1), lambda i: (0, i, 0)),
            pl.BlockSpec((1, _BLK, 16), lambda i: (0, i, 0)),
            pl.BlockSpec((16, hc1), lambda i: (0, 0)),
            pl.BlockSpec((8, hc1), lambda i: (0, 0)),
            pl.BlockSpec((hc1, hc2), lambda i: (0, 0)),
            pl.BlockSpec((hc2, 16), lambda i: (0, 0)),
        ],
        out_specs=[
            pl.BlockSpec((nspan2, _BLK, cw2), lambda i: (0, i, 0)),
            pl.BlockSpec((_BLK, 16), lambda i: (i, 0)),
        ],
        out_shape=[
            jax.ShapeDtypeStruct((nspan2, _N, cw2), jnp.float32),
            jax.ShapeDtypeStruct((_N, 16), jnp.float32),
        ],
    )(accs, dens, Eh, b, W2, AC2)


def _tc_fin(accs, dens, Eh, b, hc, nspan):
    def body(acc_ref, den_ref, eh_ref, b_ref, out_ref):
        cat = jnp.concatenate([acc_ref[q] for q in range(nspan)], axis=1)
        dexp = jnp.dot(den_ref[0], eh_ref[...],
                       preferred_element_type=jnp.float32)
        out_ref[...] = cat / (dexp + 1e-16) + b_ref[0:1, :]

    return pl.pallas_call(
        body,
        grid=(_GRID,),
        in_specs=[
            pl.BlockSpec((nspan, _BLK, hc // nspan), lambda i: (0, i, 0)),
            pl.BlockSpec((1, _BLK, 16), lambda i: (0, i, 0)),
            pl.BlockSpec((16, hc), lambda i: (0, 0)),
            pl.BlockSpec((8, hc), lambda i: (0, 0)),
        ],
        out_specs=pl.BlockSpec((_BLK, hc), lambda i: (i, 0)),
        out_shape=jax.ShapeDtypeStruct((_N, hc), jnp.float32),
    )(accs, dens, Eh, b)


def _att_matrix(att_src, att_dst):
    h, cdim = att_src.shape[1], att_src.shape[2]
    eye = jnp.eye(h, dtype=jnp.float32)
    msrc = (att_src[0][:, :, None] * eye[:, None, :]).reshape(h * cdim, h)
    mdst = (att_dst[0][:, :, None] * eye[:, None, :]).reshape(h * cdim, h)
    return jnp.concatenate([msrc, mdst], axis=1)


def _head_expander(hc):
    top = jnp.kron(jnp.eye(_H, dtype=jnp.float32),
                   jnp.ones((1, hc // _H), jnp.float32))
    return jnp.concatenate([top, jnp.zeros((_H, hc), jnp.float32)], axis=0)


def kernel(x, edge_index, W1, att_src1, att_dst1, b1,
           W2, att_src2, att_dst2, b2):
    hc1 = W1.shape[1]
    hc2 = W2.shape[1]
    c1 = hc1 // _H
    c2 = hc2 // _H

    pad = _EPAD - _E
    fill = jnp.full((pad,), _N, jnp.int32)
    srcs2 = jnp.concatenate([edge_index[0], fill]).reshape(-1, _CH)
    dsts2 = jnp.concatenate([edge_index[1], fill]).reshape(-1, _CH)

    AC1 = _att_matrix(att_src1, att_dst1)
    AC2 = _att_matrix(att_src2, att_dst2)
    Eh1 = _head_expander(hc1)
    Eh2 = _head_expander(hc2)
    b1r = jnp.broadcast_to(b1.reshape(1, hc1), (8, hc1))
    b2r = jnp.broadcast_to(b2.reshape(1, hc2), (8, hc2))

    def extend(acomb, xlq):
        nspan, _, cw = xlq.shape
        acomb_e = jnp.concatenate(
            [acomb, jnp.full((_NT - _N, 16), -1e30, jnp.float32)], axis=0)
        xlq_e = jnp.concatenate(
            [xlq, jnp.zeros((nspan, _NT - _N, cw), jnp.float32)], axis=1)
        return acomb_e, xlq_e

    np1, np2 = 2, 4
    xlq1, acomb1 = _tc_prep(x, W1, AC1, hc1, 2 * np1)
    acomb1, xlq1 = extend(acomb1, xlq1)
    accs1, dens1 = _sc_edge_layer(srcs2, dsts2, acomb1, xlq1,
                                  hc1 // (2 * np1), c1, np1)
    xlq2, acomb2 = _tc_mid(accs1, dens1, Eh1, b1r, W2, AC2, hc1, hc2,
                           2 * np1, 2 * np2)
    acomb2, xlq2 = extend(acomb2, xlq2)
    accs2, dens2 = _sc_edge_layer(srcs2, dsts2, acomb2, xlq2,
                                  hc2 // (2 * np2), c2, np2)
    return _tc_fin(accs2, dens2, Eh2, b2r, hc2, 2 * np2)
